# baseline (speedup 1.0000x reference)
import functools
import math

import jax
import jax.numpy as jnp
from jax.experimental import pallas as pl
from jax.experimental.pallas import tpu as pltpu


def _router_body(num_experts, capacity, x_ref, wr_ref, mc_ref, mr_ref, pc_ref,
                 aux_ref, cnt_ref):
    n = x_ref.shape[0]
    e_dim = num_experts
    x = x_ref[...]
    wr = wr_ref[...]
    logits = jnp.dot(x, wr, preferred_element_type=jnp.float32)
    mx = jnp.max(logits, axis=-1, keepdims=True)
    ex = jnp.exp(logits - mx)
    probs = ex / jnp.sum(ex, axis=-1, keepdims=True)

    iota_e = jax.lax.broadcasted_iota(jnp.int32, (n, e_dim), 1)
    m1 = jnp.max(probs, axis=-1, keepdims=True)
    e1 = jnp.min(jnp.where(probs == m1, iota_e, e_dim), axis=-1, keepdims=True)
    probs2 = jnp.where(iota_e == e1, -1.0, probs)
    m2 = jnp.max(probs2, axis=-1, keepdims=True)
    e2 = jnp.min(jnp.where(probs2 == m2, iota_e, e_dim), axis=-1, keepdims=True)
    s = m1 + m2
    p1 = m1 / s
    p2 = m2 / s

    imp = jnp.sum(probs, axis=0, keepdims=True)
    load = jnp.sum((iota_e == e1).astype(jnp.float32)
                   + (iota_e == e2).astype(jnp.float32), axis=0, keepdims=True)
    aux_ref[...] = (e_dim * jnp.sum(imp * load, keepdims=True)) / (n * 2)
    cnt_ref[...] = jnp.minimum(load, float(capacity)).astype(jnp.int32)

    e1f = e1.astype(jnp.float32)
    e2f = e2.astype(jnp.float32)
    zeros4 = jnp.zeros((n, 4), jnp.float32)
    packed = jnp.concatenate([e1f, e2f, p1, p2, zeros4], axis=1)
    packed_t = packed.T
    e1r = packed_t[0:1, :].astype(jnp.int32)
    e2r = packed_t[1:2, :].astype(jnp.int32)
    p1r = packed_t[2:3, :]
    p2r = packed_t[3:4, :]

    ch = 512
    tok_row = jax.lax.broadcasted_iota(jnp.int32, (1, n), 1)
    r1_parts = []
    r2_parts = []
    for c in range(n // ch):
        sl = slice(c * ch, (c + 1) * ch)
        tcol = jax.lax.broadcasted_iota(jnp.int32, (ch, 1), 0) + c * ch
        for k, (ei, pi) in enumerate(((e1[sl], p1[sl]), (e2[sl], p2[sl]))):
            cnt = jnp.zeros((ch, 1), jnp.int32)
            for kj, (ejr, pjr) in enumerate(((e1r, p1r), (e2r, p2r))):
                same = ejr == ei
                if kj < k:
                    earlier = tok_row <= tcol
                else:
                    earlier = tok_row < tcol
                beat = (pjr > pi) | ((pjr == pi) & earlier)
                cnt = cnt + jnp.sum((same & beat).astype(jnp.int32), axis=1,
                                    keepdims=True)
            if k == 0:
                r1_parts.append(cnt)
            else:
                r2_parts.append(cnt)
    r1 = jnp.concatenate(r1_parts, axis=0)
    r2 = jnp.concatenate(r2_parts, axis=0)

    zeros4i = jnp.zeros((n, 4), jnp.int32)
    mc_ref[...] = jnp.concatenate([e1, e2, r1, r2, zeros4i], axis=1)
    packed2 = jnp.concatenate([e1f, e2f, r1.astype(jnp.float32),
                               r2.astype(jnp.float32), zeros4], axis=1)
    mr_ref[...] = packed2.T.astype(jnp.int32)
    pc_ref[...] = jnp.concatenate([p1, p2, zeros4, jnp.zeros((n, 2),
                                                             jnp.float32)],
                                  axis=1)


def _ffn_body(capacity, nt, cnt_ref, x_ref, wg_ref, wu_ref, wd_ref, mr_ref,
              d_ref, g_ref, seg_ref):
    e = pl.program_id(0)
    f = pl.program_id(1)
    n = x_ref.shape[0]
    ch = 512
    mch = 128
    cnt = cnt_ref[e]

    @pl.when(f == 0)
    def _build_seg():
        mr = mr_ref[...]
        for c in range(n // ch):
            sl = slice(c * ch, (c + 1) * ch)
            e1r = mr[0:1, sl]
            e2r = mr[1:2, sl]
            r1r = mr[2:3, sl]
            r2r = mr[3:4, sl]
            iota_r = jax.lax.broadcasted_iota(jnp.int32, (capacity, ch), 0)
            g_ref[:, sl] = (
                ((e1r == e) & (iota_r == r1r)).astype(jnp.bfloat16)
                + ((e2r == e) & (iota_r == r2r)).astype(jnp.bfloat16))
        d_ref[0] = jnp.zeros((capacity, x_ref.shape[1]), jnp.bfloat16)
        for m in range(capacity // mch):
            msl = slice(m * mch, (m + 1) * mch)

            @pl.when(cnt > m * mch)
            def _():
                seg32 = jnp.dot(g_ref[msl, :], x_ref[...],
                                preferred_element_type=jnp.float32)
                seg_ref[msl, :] = seg32.astype(jnp.bfloat16)

    wgb = wg_ref[0].astype(jnp.bfloat16)
    wub = wu_ref[0].astype(jnp.bfloat16)
    wdb = wd_ref[0].astype(jnp.bfloat16)
    for m in range(capacity // mch):
        msl = slice(m * mch, (m + 1) * mch)

        @pl.when(cnt > m * mch)
        def _():
            gate = jnp.dot(seg_ref[msl, :], wgb,
                           preferred_element_type=jnp.float32)
            up = jnp.dot(seg_ref[msl, :], wub,
                         preferred_element_type=jnp.float32)
            h = (gate * jax.nn.sigmoid(gate)) * up
            dpart = jnp.dot(h.astype(jnp.bfloat16), wdb,
                            preferred_element_type=jnp.float32)
            d_ref[0, msl, :] = (d_ref[0, msl, :].astype(jnp.float32)
                                + dpart).astype(jnp.bfloat16)


def _combine_body(capacity, num_experts, mc_ref, pc_ref, d_ref, out_ref,
                  w_ref):
    ch = out_ref.shape[0]
    hidden = out_ref.shape[1]
    mc = mc_ref[...]
    pc = pc_ref[...]
    e1c = mc[:, 0:1]
    e2c = mc[:, 1:2]
    r1c = mc[:, 2:3]
    r2c = mc[:, 3:4]
    p1c = pc[:, 0:1]
    p2c = pc[:, 1:2]
    iota_rw = jax.lax.broadcasted_iota(jnp.int32, (ch, capacity), 1)
    acc = jnp.zeros((ch, hidden), jnp.float32)
    for e in range(num_experts):
        w = (jnp.where((e1c == e) & (iota_rw == r1c), p1c, 0.0)
             + jnp.where((e2c == e) & (iota_rw == r2c), p2c, 0.0))
        w_ref[...] = w.astype(jnp.bfloat16)
        acc = acc + jnp.dot(w_ref[...], d_ref[e],
                            preferred_element_type=jnp.float32)
    out_ref[...] = acc


def kernel(hidden_states, w_router, w_gate, w_up, w_down):
    n, hidden = hidden_states.shape
    num_experts = w_router.shape[1]
    ffn = w_gate.shape[2]
    capacity = max(4, math.ceil(1.25 * n * 2 / num_experts))
    capacity = min(capacity, n * 2)

    mc, mr, pc, aux, cnts = pl.pallas_call(
        functools.partial(_router_body, num_experts, capacity),
        in_specs=[
            pl.BlockSpec((n, hidden), lambda: (0, 0)),
            pl.BlockSpec((hidden, num_experts), lambda: (0, 0)),
        ],
        out_specs=[
            pl.BlockSpec((n, 8), lambda: (0, 0)),
            pl.BlockSpec((8, n), lambda: (0, 0)),
            pl.BlockSpec((n, 8), lambda: (0, 0)),
            pl.BlockSpec((1, 1), lambda: (0, 0)),
            pl.BlockSpec((1, num_experts), lambda: (0, 0)),
        ],
        out_shape=[
            jax.ShapeDtypeStruct((n, 8), jnp.int32),
            jax.ShapeDtypeStruct((8, n), jnp.int32),
            jax.ShapeDtypeStruct((n, 8), jnp.float32),
            jax.ShapeDtypeStruct((1, 1), jnp.float32),
            jax.ShapeDtypeStruct((1, num_experts), jnp.int32),
        ],
    )(hidden_states, w_router)

    x_bf = hidden_states.astype(jnp.bfloat16)

    tile = 512
    nt = ffn // tile
    d_all = pl.pallas_call(
        functools.partial(_ffn_body, capacity, nt),
        grid_spec=pltpu.PrefetchScalarGridSpec(
            num_scalar_prefetch=1,
            grid=(num_experts, nt),
            in_specs=[
                pl.BlockSpec((n, hidden), lambda e, f, c: (0, 0)),
                pl.BlockSpec((1, hidden, tile), lambda e, f, c: (e, 0, f)),
                pl.BlockSpec((1, hidden, tile), lambda e, f, c: (e, 0, f)),
                pl.BlockSpec((1, tile, hidden), lambda e, f, c: (e, f, 0)),
                pl.BlockSpec((8, n), lambda e, f, c: (0, 0)),
            ],
            out_specs=pl.BlockSpec((1, capacity, hidden),
                                   lambda e, f, c: (e, 0, 0)),
            scratch_shapes=[
                pltpu.VMEM((capacity, n), jnp.bfloat16),
                pltpu.VMEM((capacity, hidden), jnp.bfloat16),
            ],
        ),
        out_shape=jax.ShapeDtypeStruct((num_experts, capacity, hidden),
                                       jnp.bfloat16),
    )(cnts.reshape(num_experts), x_bf, w_gate, w_up, w_down, mr)

    cch = 512
    out = pl.pallas_call(
        functools.partial(_combine_body, capacity, num_experts),
        grid=(n // cch,),
        in_specs=[
            pl.BlockSpec((cch, 8), lambda c: (c, 0)),
            pl.BlockSpec((cch, 8), lambda c: (c, 0)),
            pl.BlockSpec((num_experts, capacity, hidden), lambda c: (0, 0, 0)),
        ],
        out_specs=pl.BlockSpec((cch, hidden), lambda c: (c, 0)),
        out_shape=jax.ShapeDtypeStruct((n, hidden), jnp.float32),
        scratch_shapes=[
            pltpu.VMEM((cch, capacity), jnp.bfloat16),
        ],
    )(mc, pc, d_all)

    return out, aux[0, 0]


# speedup vs baseline: 1.1537x; 1.1537x over previous
import functools
import math

import jax
import jax.numpy as jnp
from jax.experimental import pallas as pl
from jax.experimental.pallas import tpu as pltpu
from jax.experimental.pallas import tpu_sc as plsc


def _sc_dispatch(num_experts, capacity, n, mc):
    nslots = num_experts * capacity
    chunk = 512

    @pl.kernel(
        out_type=jax.ShapeDtypeStruct((nslots,), jnp.int32),
        mesh=plsc.ScalarSubcoreMesh(axis_name="core", num_cores=2),
        scratch_types=[
            pltpu.SMEM((chunk * 8,), jnp.int32),
            pltpu.SMEM((nslots,), jnp.int32),
            pltpu.SemaphoreType.DMA,
        ],
    )
    def dispatch(mc_hbm, out_hbm, mc_smem, buf_smem, sem):
        cid = jax.lax.axis_index("core")

        @pl.when(cid == 0)
        def _():
            @pl.loop(0, nslots)
            def _(i):
                buf_smem[i] = 0

            for c in range(n // chunk):
                pltpu.async_copy(
                    mc_hbm.at[pl.ds(c * chunk * 8, chunk * 8)], mc_smem, sem
                ).wait()

                @pl.loop(0, chunk)
                def _(i):
                    e1 = mc_smem[i * 8 + 0]
                    e2 = mc_smem[i * 8 + 1]
                    r1 = mc_smem[i * 8 + 2]
                    r2 = mc_smem[i * 8 + 3]

                    @pl.when(r1 < capacity)
                    def _():
                        buf_smem[e1 * capacity + r1] = c * chunk + i

                    @pl.when(r2 < capacity)
                    def _():
                        buf_smem[e2 * capacity + r2] = c * chunk + i

            pltpu.async_copy(buf_smem, out_hbm, sem).wait()

    return dispatch(mc.reshape(-1))


def _router_body(num_experts, capacity, x_ref, wr_ref, mc_ref, mr_ref, pc_ref,
                 aux_ref):
    n = x_ref.shape[0]
    e_dim = num_experts
    x = x_ref[...]
    wr = wr_ref[...]
    logits = jnp.dot(x, wr, preferred_element_type=jnp.float32)
    mx = jnp.max(logits, axis=-1, keepdims=True)
    ex = jnp.exp(logits - mx)
    probs = ex / jnp.sum(ex, axis=-1, keepdims=True)

    iota_e = jax.lax.broadcasted_iota(jnp.int32, (n, e_dim), 1)
    m1 = jnp.max(probs, axis=-1, keepdims=True)
    e1 = jnp.min(jnp.where(probs == m1, iota_e, e_dim), axis=-1, keepdims=True)
    probs2 = jnp.where(iota_e == e1, -1.0, probs)
    m2 = jnp.max(probs2, axis=-1, keepdims=True)
    e2 = jnp.min(jnp.where(probs2 == m2, iota_e, e_dim), axis=-1, keepdims=True)
    s = m1 + m2
    p1 = m1 / s
    p2 = m2 / s

    imp = jnp.sum(probs, axis=0, keepdims=True)
    load = jnp.sum((iota_e == e1).astype(jnp.float32)
                   + (iota_e == e2).astype(jnp.float32), axis=0, keepdims=True)
    aux_ref[...] = (e_dim * jnp.sum(imp * load, keepdims=True)) / (n * 2)

    e1f = e1.astype(jnp.float32)
    e2f = e2.astype(jnp.float32)
    zeros4 = jnp.zeros((n, 4), jnp.float32)
    packed = jnp.concatenate([e1f, e2f, p1, p2, zeros4], axis=1)
    packed_t = packed.T
    e1r = packed_t[0:1, :].astype(jnp.int32)
    e2r = packed_t[1:2, :].astype(jnp.int32)
    p1r = packed_t[2:3, :]
    p2r = packed_t[3:4, :]

    ch = 512
    tok_row = jax.lax.broadcasted_iota(jnp.int32, (1, n), 1)
    r1_parts = []
    r2_parts = []
    for c in range(n // ch):
        sl = slice(c * ch, (c + 1) * ch)
        tcol = jax.lax.broadcasted_iota(jnp.int32, (ch, 1), 0) + c * ch
        for k, (ei, pi) in enumerate(((e1[sl], p1[sl]), (e2[sl], p2[sl]))):
            cnt = jnp.zeros((ch, 1), jnp.int32)
            for kj, (ejr, pjr) in enumerate(((e1r, p1r), (e2r, p2r))):
                same = ejr == ei
                if kj < k:
                    earlier = tok_row <= tcol
                else:
                    earlier = tok_row < tcol
                beat = (pjr > pi) | ((pjr == pi) & earlier)
                cnt = cnt + jnp.sum((same & beat).astype(jnp.int32), axis=1,
                                    keepdims=True)
            if k == 0:
                r1_parts.append(cnt)
            else:
                r2_parts.append(cnt)
    r1 = jnp.concatenate(r1_parts, axis=0)
    r2 = jnp.concatenate(r2_parts, axis=0)

    zeros4i = jnp.zeros((n, 4), jnp.int32)
    mc_ref[...] = jnp.concatenate([e1, e2, r1, r2, zeros4i], axis=1)
    packed2 = jnp.concatenate([e1f, e2f, r1.astype(jnp.float32),
                               r2.astype(jnp.float32), zeros4], axis=1)
    mr_ref[...] = packed2.T.astype(jnp.int32)
    pc_ref[...] = jnp.concatenate([p1, p2, zeros4, jnp.zeros((n, 2),
                                                             jnp.float32)],
                                  axis=1)


def _ffn_body(capacity, nt, x_ref, wg_ref, wu_ref, wd_ref, st_ref, d_ref,
              g_ref, seg_ref):
    e = pl.program_id(0)
    f = pl.program_id(1)
    n = x_ref.shape[0]
    ch = 512

    @pl.when(f == 0)
    def _build_seg():
        tok = st_ref[0]
        for c in range(n // ch):
            sl = slice(c * ch, (c + 1) * ch)
            iota_t = (jax.lax.broadcasted_iota(jnp.int32, (capacity, ch), 1)
                      + c * ch)
            g_ref[:, sl] = (tok == iota_t).astype(jnp.bfloat16)
        seg32 = jnp.dot(g_ref[...], x_ref[...],
                        preferred_element_type=jnp.float32)
        seg_ref[...] = seg32.astype(jnp.bfloat16)

    gate = jnp.dot(seg_ref[...], wg_ref[0].astype(jnp.bfloat16),
                   preferred_element_type=jnp.float32)
    up = jnp.dot(seg_ref[...], wu_ref[0].astype(jnp.bfloat16),
                 preferred_element_type=jnp.float32)
    h = (gate * jax.nn.sigmoid(gate)) * up
    dpart = jnp.dot(h.astype(jnp.bfloat16), wd_ref[0].astype(jnp.bfloat16),
                    preferred_element_type=jnp.float32)

    @pl.when(f == 0)
    def _init_d():
        d_ref[0] = dpart.astype(jnp.bfloat16)

    @pl.when(f > 0)
    def _acc_d():
        d_ref[0] = (d_ref[0].astype(jnp.float32) + dpart).astype(jnp.bfloat16)


def _combine_body(capacity, num_experts, mc_ref, pc_ref, d_ref, out_ref,
                  w_ref):
    ch = out_ref.shape[0]
    hidden = out_ref.shape[1]
    mc = mc_ref[...]
    pc = pc_ref[...]
    e1c = mc[:, 0:1]
    e2c = mc[:, 1:2]
    r1c = mc[:, 2:3]
    r2c = mc[:, 3:4]
    p1c = pc[:, 0:1]
    p2c = pc[:, 1:2]
    iota_rw = jax.lax.broadcasted_iota(jnp.int32, (ch, capacity), 1)
    acc = jnp.zeros((ch, hidden), jnp.float32)
    for e in range(num_experts):
        w = (jnp.where((e1c == e) & (iota_rw == r1c), p1c, 0.0)
             + jnp.where((e2c == e) & (iota_rw == r2c), p2c, 0.0))
        w_ref[...] = w.astype(jnp.bfloat16)
        acc = acc + jnp.dot(w_ref[...], d_ref[e],
                            preferred_element_type=jnp.float32)
    out_ref[...] = acc


def kernel(hidden_states, w_router, w_gate, w_up, w_down):
    n, hidden = hidden_states.shape
    num_experts = w_router.shape[1]
    ffn = w_gate.shape[2]
    capacity = max(4, math.ceil(1.25 * n * 2 / num_experts))
    capacity = min(capacity, n * 2)

    mc, mr, pc, aux = pl.pallas_call(
        functools.partial(_router_body, num_experts, capacity),
        in_specs=[
            pl.BlockSpec((n, hidden), lambda: (0, 0)),
            pl.BlockSpec((hidden, num_experts), lambda: (0, 0)),
        ],
        out_specs=[
            pl.BlockSpec((n, 8), lambda: (0, 0)),
            pl.BlockSpec((8, n), lambda: (0, 0)),
            pl.BlockSpec((n, 8), lambda: (0, 0)),
            pl.BlockSpec((1, 1), lambda: (0, 0)),
        ],
        out_shape=[
            jax.ShapeDtypeStruct((n, 8), jnp.int32),
            jax.ShapeDtypeStruct((8, n), jnp.int32),
            jax.ShapeDtypeStruct((n, 8), jnp.float32),
            jax.ShapeDtypeStruct((1, 1), jnp.float32),
        ],
    )(hidden_states, w_router)

    srctok = _sc_dispatch(num_experts, capacity, n, mc)
    srctok = srctok.reshape(num_experts, capacity, 1)

    x_bf = hidden_states.astype(jnp.bfloat16)

    tile = 512
    nt = ffn // tile
    d_all = pl.pallas_call(
        functools.partial(_ffn_body, capacity, nt),
        grid=(num_experts, nt),
        in_specs=[
            pl.BlockSpec((n, hidden), lambda e, f: (0, 0)),
            pl.BlockSpec((1, hidden, tile), lambda e, f: (e, 0, f)),
            pl.BlockSpec((1, hidden, tile), lambda e, f: (e, 0, f)),
            pl.BlockSpec((1, tile, hidden), lambda e, f: (e, f, 0)),
            pl.BlockSpec((1, capacity, 1), lambda e, f: (e, 0, 0)),
        ],
        out_specs=pl.BlockSpec((1, capacity, hidden), lambda e, f: (e, 0, 0)),
        out_shape=jax.ShapeDtypeStruct((num_experts, capacity, hidden),
                                       jnp.bfloat16),
        scratch_shapes=[
            pltpu.VMEM((capacity, n), jnp.bfloat16),
            pltpu.VMEM((capacity, hidden), jnp.bfloat16),
        ],
    )(x_bf, w_gate, w_up, w_down, srctok)

    cch = 512
    out = pl.pallas_call(
        functools.partial(_combine_body, capacity, num_experts),
        grid=(n // cch,),
        in_specs=[
            pl.BlockSpec((cch, 8), lambda c: (c, 0)),
            pl.BlockSpec((cch, 8), lambda c: (c, 0)),
            pl.BlockSpec((num_experts, capacity, hidden), lambda c: (0, 0, 0)),
        ],
        out_specs=pl.BlockSpec((cch, hidden), lambda c: (c, 0)),
        out_shape=jax.ShapeDtypeStruct((n, hidden), jnp.float32),
        scratch_shapes=[
            pltpu.VMEM((cch, capacity), jnp.bfloat16),
        ],
    )(mc, pc, d_all)

    return out, aux[0, 0]


# speedup vs baseline: 1.1782x; 1.0212x over previous
import functools
import math

import jax
import jax.numpy as jnp
from jax.experimental import pallas as pl
from jax.experimental.pallas import tpu as pltpu
from jax.experimental.pallas import tpu_sc as plsc


def _sc_dispatch(num_experts, capacity, n, mc):
    nslots = num_experts * capacity
    chunk = 512

    @pl.kernel(
        out_type=jax.ShapeDtypeStruct((nslots,), jnp.int32),
        mesh=plsc.ScalarSubcoreMesh(axis_name="core", num_cores=2),
        scratch_types=[
            pltpu.SMEM((chunk * 8,), jnp.int32),
            pltpu.SMEM((nslots,), jnp.int32),
            pltpu.SemaphoreType.DMA,
        ],
    )
    def dispatch(mc_hbm, out_hbm, mc_smem, buf_smem, sem):
        cid = jax.lax.axis_index("core")

        @pl.when(cid == 0)
        def _():
            for c in range(n // chunk):
                pltpu.async_copy(
                    mc_hbm.at[pl.ds(c * chunk * 8, chunk * 8)], mc_smem, sem
                ).wait()

                @pl.loop(0, chunk)
                def _(i):
                    e1 = mc_smem[i * 8 + 0]
                    e2 = mc_smem[i * 8 + 1]
                    r1 = mc_smem[i * 8 + 2]
                    r2 = mc_smem[i * 8 + 3]

                    @pl.when(r1 < capacity)
                    def _():
                        buf_smem[e1 * capacity + r1] = c * chunk + i

                    @pl.when(r2 < capacity)
                    def _():
                        buf_smem[e2 * capacity + r2] = c * chunk + i

            pltpu.async_copy(buf_smem, out_hbm, sem).wait()

    return dispatch(mc.reshape(-1))


def _router_body(num_experts, capacity, x_ref, wr_ref, mc_ref, mr_ref, pc_ref,
                 aux_ref):
    n = x_ref.shape[0]
    e_dim = num_experts
    x = x_ref[...]
    wr = wr_ref[...]
    logits = jnp.dot(x, wr, preferred_element_type=jnp.float32)
    mx = jnp.max(logits, axis=-1, keepdims=True)
    ex = jnp.exp(logits - mx)
    probs = ex / jnp.sum(ex, axis=-1, keepdims=True)

    iota_e = jax.lax.broadcasted_iota(jnp.int32, (n, e_dim), 1)
    m1 = jnp.max(probs, axis=-1, keepdims=True)
    e1 = jnp.min(jnp.where(probs == m1, iota_e, e_dim), axis=-1, keepdims=True)
    probs2 = jnp.where(iota_e == e1, -1.0, probs)
    m2 = jnp.max(probs2, axis=-1, keepdims=True)
    e2 = jnp.min(jnp.where(probs2 == m2, iota_e, e_dim), axis=-1, keepdims=True)
    s = m1 + m2
    p1 = m1 / s
    p2 = m2 / s

    imp = jnp.sum(probs, axis=0, keepdims=True)
    load = jnp.sum((iota_e == e1).astype(jnp.float32)
                   + (iota_e == e2).astype(jnp.float32), axis=0, keepdims=True)
    aux_ref[...] = (e_dim * jnp.sum(imp * load, keepdims=True)) / (n * 2)

    e1f = e1.astype(jnp.float32)
    e2f = e2.astype(jnp.float32)
    zeros4 = jnp.zeros((n, 4), jnp.float32)
    packed = jnp.concatenate([e1f, e2f, p1, p2, zeros4], axis=1)
    packed_t = packed.T
    e1r = packed_t[0:1, :].astype(jnp.int32)
    e2r = packed_t[1:2, :].astype(jnp.int32)
    p1r = packed_t[2:3, :]
    p2r = packed_t[3:4, :]

    ch = 512
    tok_row = jax.lax.broadcasted_iota(jnp.int32, (1, n), 1)
    r1_parts = []
    r2_parts = []
    for c in range(n // ch):
        sl = slice(c * ch, (c + 1) * ch)
        tcol = jax.lax.broadcasted_iota(jnp.int32, (ch, 1), 0) + c * ch
        for k, (ei, pi) in enumerate(((e1[sl], p1[sl]), (e2[sl], p2[sl]))):
            cnt = jnp.zeros((ch, 1), jnp.int32)
            for kj, (ejr, pjr) in enumerate(((e1r, p1r), (e2r, p2r))):
                same = ejr == ei
                if kj < k:
                    earlier = tok_row <= tcol
                else:
                    earlier = tok_row < tcol
                beat = (pjr > pi) | ((pjr == pi) & earlier)
                cnt = cnt + jnp.sum((same & beat).astype(jnp.int32), axis=1,
                                    keepdims=True)
            if k == 0:
                r1_parts.append(cnt)
            else:
                r2_parts.append(cnt)
    r1 = jnp.concatenate(r1_parts, axis=0)
    r2 = jnp.concatenate(r2_parts, axis=0)

    zeros4i = jnp.zeros((n, 4), jnp.int32)
    mc_ref[...] = jnp.concatenate([e1, e2, r1, r2, zeros4i], axis=1)
    packed2 = jnp.concatenate([e1f, e2f, r1.astype(jnp.float32),
                               r2.astype(jnp.float32), zeros4], axis=1)
    mr_ref[...] = packed2.T.astype(jnp.int32)
    pc_ref[...] = jnp.concatenate([p1, p2, zeros4, jnp.zeros((n, 2),
                                                             jnp.float32)],
                                  axis=1)


def _ffn_body(capacity, nt, x_ref, wg_ref, wu_ref, wd_ref, st_ref, d_ref,
              g_ref, seg_ref):
    e = pl.program_id(0)
    f = pl.program_id(1)
    n = x_ref.shape[0]
    ch = 512

    @pl.when(f == 0)
    def _build_seg():
        tok = st_ref[0]
        for c in range(n // ch):
            sl = slice(c * ch, (c + 1) * ch)
            iota_t = (jax.lax.broadcasted_iota(jnp.int32, (capacity, ch), 1)
                      + c * ch)
            g_ref[:, sl] = (tok == iota_t).astype(jnp.bfloat16)
        seg32 = jnp.dot(g_ref[...], x_ref[...],
                        preferred_element_type=jnp.float32)
        seg_ref[...] = seg32.astype(jnp.bfloat16)

    gate = jnp.dot(seg_ref[...], wg_ref[0].astype(jnp.bfloat16),
                   preferred_element_type=jnp.float32)
    up = jnp.dot(seg_ref[...], wu_ref[0].astype(jnp.bfloat16),
                 preferred_element_type=jnp.float32)
    h = (gate * jax.nn.sigmoid(gate)) * up
    dpart = jnp.dot(h.astype(jnp.bfloat16), wd_ref[0].astype(jnp.bfloat16),
                    preferred_element_type=jnp.float32)

    @pl.when(f == 0)
    def _init_d():
        d_ref[0] = dpart.astype(jnp.bfloat16)

    @pl.when(f > 0)
    def _acc_d():
        d_ref[0] = (d_ref[0].astype(jnp.float32) + dpart).astype(jnp.bfloat16)


def _combine_body(capacity, num_experts, mc_ref, pc_ref, d_ref, out_ref,
                  w_ref):
    ch = out_ref.shape[0]
    hidden = out_ref.shape[1]
    mc = mc_ref[...]
    pc = pc_ref[...]
    e1c = mc[:, 0:1]
    e2c = mc[:, 1:2]
    r1c = mc[:, 2:3]
    r2c = mc[:, 3:4]
    p1c = pc[:, 0:1]
    p2c = pc[:, 1:2]
    iota_rw = jax.lax.broadcasted_iota(jnp.int32, (ch, capacity), 1)
    acc = jnp.zeros((ch, hidden), jnp.float32)
    for e in range(num_experts):
        w = (jnp.where((e1c == e) & (iota_rw == r1c), p1c, 0.0)
             + jnp.where((e2c == e) & (iota_rw == r2c), p2c, 0.0))
        w_ref[...] = w.astype(jnp.bfloat16)
        acc = acc + jnp.dot(w_ref[...], d_ref[e],
                            preferred_element_type=jnp.float32)
    out_ref[...] = acc


def kernel(hidden_states, w_router, w_gate, w_up, w_down):
    n, hidden = hidden_states.shape
    num_experts = w_router.shape[1]
    ffn = w_gate.shape[2]
    capacity = max(4, math.ceil(1.25 * n * 2 / num_experts))
    capacity = min(capacity, n * 2)

    mc, mr, pc, aux = pl.pallas_call(
        functools.partial(_router_body, num_experts, capacity),
        in_specs=[
            pl.BlockSpec((n, hidden), lambda: (0, 0)),
            pl.BlockSpec((hidden, num_experts), lambda: (0, 0)),
        ],
        out_specs=[
            pl.BlockSpec((n, 8), lambda: (0, 0)),
            pl.BlockSpec((8, n), lambda: (0, 0)),
            pl.BlockSpec((n, 8), lambda: (0, 0)),
            pl.BlockSpec((1, 1), lambda: (0, 0)),
        ],
        out_shape=[
            jax.ShapeDtypeStruct((n, 8), jnp.int32),
            jax.ShapeDtypeStruct((8, n), jnp.int32),
            jax.ShapeDtypeStruct((n, 8), jnp.float32),
            jax.ShapeDtypeStruct((1, 1), jnp.float32),
        ],
    )(hidden_states, w_router)

    srctok = _sc_dispatch(num_experts, capacity, n, mc)
    srctok = srctok.reshape(num_experts, capacity, 1)

    x_bf = hidden_states.astype(jnp.bfloat16)

    tile = 512
    nt = ffn // tile
    d_all = pl.pallas_call(
        functools.partial(_ffn_body, capacity, nt),
        grid=(num_experts, nt),
        in_specs=[
            pl.BlockSpec((n, hidden), lambda e, f: (0, 0)),
            pl.BlockSpec((1, hidden, tile), lambda e, f: (e, 0, f)),
            pl.BlockSpec((1, hidden, tile), lambda e, f: (e, 0, f)),
            pl.BlockSpec((1, tile, hidden), lambda e, f: (e, f, 0)),
            pl.BlockSpec((1, capacity, 1), lambda e, f: (e, 0, 0)),
        ],
        out_specs=pl.BlockSpec((1, capacity, hidden), lambda e, f: (e, 0, 0)),
        out_shape=jax.ShapeDtypeStruct((num_experts, capacity, hidden),
                                       jnp.bfloat16),
        scratch_shapes=[
            pltpu.VMEM((capacity, n), jnp.bfloat16),
            pltpu.VMEM((capacity, hidden), jnp.bfloat16),
        ],
    )(x_bf, w_gate, w_up, w_down, srctok)

    cch = 512
    out = pl.pallas_call(
        functools.partial(_combine_body, capacity, num_experts),
        grid=(n // cch,),
        in_specs=[
            pl.BlockSpec((cch, 8), lambda c: (c, 0)),
            pl.BlockSpec((cch, 8), lambda c: (c, 0)),
            pl.BlockSpec((num_experts, capacity, hidden), lambda c: (0, 0, 0)),
        ],
        out_specs=pl.BlockSpec((cch, hidden), lambda c: (c, 0)),
        out_shape=jax.ShapeDtypeStruct((n, hidden), jnp.float32),
        scratch_shapes=[
            pltpu.VMEM((cch, capacity), jnp.bfloat16),
        ],
    )(mc, pc, d_all)

    return out, aux[0, 0]


# speedup vs baseline: 1.2061x; 1.0237x over previous
import dataclasses
import functools
import math

import jax
import jax.numpy as jnp
from jax.experimental import pallas as pl
from jax.experimental.pallas import tpu as pltpu
from jax.experimental.pallas import tpu_sc as plsc


def _sc_dispatch(num_experts, capacity, n, po):
    nslots = num_experts * capacity
    nflat = 2 * n

    cp = pltpu.CompilerParams()
    if "needs_layout_passes" in pltpu.CompilerParams.__dataclass_fields__:
        cp = dataclasses.replace(cp, needs_layout_passes=False)

    @pl.kernel(
        out_type=jax.ShapeDtypeStruct((nslots,), jnp.int32),
        compiler_params=cp,
        mesh=plsc.VectorSubcoreMesh(core_axis_name="core",
                                    subcore_axis_name="subcore"),
        scratch_types=[
            pltpu.VMEM((nflat,), jnp.int32),
            pltpu.VMEM((nslots + 16,), jnp.int32),
            pltpu.SemaphoreType.DMA,
        ],
    )
    def dispatch(po_hbm, out_hbm, pos_vmem, buf_vmem, sem):
        cid = jax.lax.axis_index("core")
        sid = jax.lax.axis_index("subcore")

        @pl.when((cid == 0) & (sid == 0))
        def _():
            pltpu.async_copy(po_hbm, pos_vmem, sem).wait()
            lane = jax.lax.iota(jnp.int32, 16)
            for i in range(nflat // 16):
                idx = pos_vmem[pl.ds(i * 16, 16)]
                tok = (lane + (i * 16)) & (n - 1)
                plsc.store_scatter(buf_vmem, [idx], tok)
            pltpu.async_copy(buf_vmem.at[pl.ds(0, nslots)], out_hbm,
                             sem).wait()

    return dispatch(po.reshape(-1))


def _router_body(num_experts, capacity, x_ref, wr_ref, mc_ref, mr_ref, pc_ref,
                 aux_ref, po_ref):
    n = x_ref.shape[0]
    e_dim = num_experts
    x = x_ref[...]
    wr = wr_ref[...]
    logits = jnp.dot(x, wr, preferred_element_type=jnp.float32)
    mx = jnp.max(logits, axis=-1, keepdims=True)
    ex = jnp.exp(logits - mx)
    probs = ex / jnp.sum(ex, axis=-1, keepdims=True)

    iota_e = jax.lax.broadcasted_iota(jnp.int32, (n, e_dim), 1)
    m1 = jnp.max(probs, axis=-1, keepdims=True)
    e1 = jnp.min(jnp.where(probs == m1, iota_e, e_dim), axis=-1, keepdims=True)
    probs2 = jnp.where(iota_e == e1, -1.0, probs)
    m2 = jnp.max(probs2, axis=-1, keepdims=True)
    e2 = jnp.min(jnp.where(probs2 == m2, iota_e, e_dim), axis=-1, keepdims=True)
    s = m1 + m2
    p1 = m1 / s
    p2 = m2 / s

    imp = jnp.sum(probs, axis=0, keepdims=True)
    load = jnp.sum((iota_e == e1).astype(jnp.float32)
                   + (iota_e == e2).astype(jnp.float32), axis=0, keepdims=True)
    aux_ref[...] = (e_dim * jnp.sum(imp * load, keepdims=True)) / (n * 2)

    e1f = e1.astype(jnp.float32)
    e2f = e2.astype(jnp.float32)
    zeros4 = jnp.zeros((n, 4), jnp.float32)
    packed = jnp.concatenate([e1f, e2f, p1, p2, zeros4], axis=1)
    packed_t = packed.T
    e1r = packed_t[0:1, :].astype(jnp.int32)
    e2r = packed_t[1:2, :].astype(jnp.int32)
    p1r = packed_t[2:3, :]
    p2r = packed_t[3:4, :]

    ch = 512
    tok_row = jax.lax.broadcasted_iota(jnp.int32, (1, n), 1)
    r1_parts = []
    r2_parts = []
    for c in range(n // ch):
        sl = slice(c * ch, (c + 1) * ch)
        tcol = jax.lax.broadcasted_iota(jnp.int32, (ch, 1), 0) + c * ch
        for k, (ei, pi) in enumerate(((e1[sl], p1[sl]), (e2[sl], p2[sl]))):
            cnt = jnp.zeros((ch, 1), jnp.int32)
            for kj, (ejr, pjr) in enumerate(((e1r, p1r), (e2r, p2r))):
                same = ejr == ei
                if kj < k:
                    earlier = tok_row <= tcol
                else:
                    earlier = tok_row < tcol
                beat = (pjr > pi) | ((pjr == pi) & earlier)
                cnt = cnt + jnp.sum((same & beat).astype(jnp.int32), axis=1,
                                    keepdims=True)
            if k == 0:
                r1_parts.append(cnt)
            else:
                r2_parts.append(cnt)
    r1 = jnp.concatenate(r1_parts, axis=0)
    r2 = jnp.concatenate(r2_parts, axis=0)

    zeros4i = jnp.zeros((n, 4), jnp.int32)
    mc_ref[...] = jnp.concatenate([e1, e2, r1, r2, zeros4i], axis=1)
    packed2 = jnp.concatenate([e1f, e2f, r1.astype(jnp.float32),
                               r2.astype(jnp.float32), zeros4], axis=1)
    mrt = packed2.T.astype(jnp.int32)
    mr_ref[...] = mrt
    nslots = num_experts * capacity
    pos1 = jnp.where(mrt[2:3, :] < capacity,
                     mrt[0:1, :] * capacity + mrt[2:3, :], nslots)
    pos2 = jnp.where(mrt[3:4, :] < capacity,
                     mrt[1:2, :] * capacity + mrt[3:4, :], nslots)
    po_ref[...] = jnp.concatenate([pos1, pos2], axis=0)
    pc_ref[...] = jnp.concatenate([p1, p2, zeros4, jnp.zeros((n, 2),
                                                             jnp.float32)],
                                  axis=1)


def _ffn_body(capacity, nt, x_ref, wg_ref, wu_ref, wd_ref, st_ref, d_ref,
              g_ref, seg_ref):
    e = pl.program_id(0)
    f = pl.program_id(1)
    n = x_ref.shape[0]
    ch = 512

    @pl.when(f == 0)
    def _build_seg():
        tok = st_ref[0]
        for c in range(n // ch):
            sl = slice(c * ch, (c + 1) * ch)
            iota_t = (jax.lax.broadcasted_iota(jnp.int32, (capacity, ch), 1)
                      + c * ch)
            g_ref[:, sl] = (tok == iota_t).astype(jnp.bfloat16)
        seg32 = jnp.dot(g_ref[...], x_ref[...],
                        preferred_element_type=jnp.float32)
        seg_ref[...] = seg32.astype(jnp.bfloat16)

    gate = jnp.dot(seg_ref[...], wg_ref[0].astype(jnp.bfloat16),
                   preferred_element_type=jnp.float32)
    up = jnp.dot(seg_ref[...], wu_ref[0].astype(jnp.bfloat16),
                 preferred_element_type=jnp.float32)
    h = (gate * jax.nn.sigmoid(gate)) * up
    dpart = jnp.dot(h.astype(jnp.bfloat16), wd_ref[0].astype(jnp.bfloat16),
                    preferred_element_type=jnp.float32)

    @pl.when(f == 0)
    def _init_d():
        d_ref[0] = dpart.astype(jnp.bfloat16)

    @pl.when(f > 0)
    def _acc_d():
        d_ref[0] = (d_ref[0].astype(jnp.float32) + dpart).astype(jnp.bfloat16)


def _combine_body(capacity, num_experts, mc_ref, pc_ref, d_ref, out_ref,
                  w_ref):
    ch = out_ref.shape[0]
    hidden = out_ref.shape[1]
    mc = mc_ref[...]
    pc = pc_ref[...]
    e1c = mc[:, 0:1]
    e2c = mc[:, 1:2]
    r1c = mc[:, 2:3]
    r2c = mc[:, 3:4]
    p1c = pc[:, 0:1]
    p2c = pc[:, 1:2]
    iota_rw = jax.lax.broadcasted_iota(jnp.int32, (ch, capacity), 1)
    acc = jnp.zeros((ch, hidden), jnp.float32)
    for e in range(num_experts):
        w = (jnp.where((e1c == e) & (iota_rw == r1c), p1c, 0.0)
             + jnp.where((e2c == e) & (iota_rw == r2c), p2c, 0.0))
        w_ref[...] = w.astype(jnp.bfloat16)
        acc = acc + jnp.dot(w_ref[...], d_ref[e],
                            preferred_element_type=jnp.float32)
    out_ref[...] = acc


def kernel(hidden_states, w_router, w_gate, w_up, w_down):
    n, hidden = hidden_states.shape
    num_experts = w_router.shape[1]
    ffn = w_gate.shape[2]
    capacity = max(4, math.ceil(1.25 * n * 2 / num_experts))
    capacity = min(capacity, n * 2)

    mc, mr, pc, aux, po = pl.pallas_call(
        functools.partial(_router_body, num_experts, capacity),
        in_specs=[
            pl.BlockSpec((n, hidden), lambda: (0, 0)),
            pl.BlockSpec((hidden, num_experts), lambda: (0, 0)),
        ],
        out_specs=[
            pl.BlockSpec((n, 8), lambda: (0, 0)),
            pl.BlockSpec((8, n), lambda: (0, 0)),
            pl.BlockSpec((n, 8), lambda: (0, 0)),
            pl.BlockSpec((1, 1), lambda: (0, 0)),
            pl.BlockSpec((2, n), lambda: (0, 0)),
        ],
        out_shape=[
            jax.ShapeDtypeStruct((n, 8), jnp.int32),
            jax.ShapeDtypeStruct((8, n), jnp.int32),
            jax.ShapeDtypeStruct((n, 8), jnp.float32),
            jax.ShapeDtypeStruct((1, 1), jnp.float32),
            jax.ShapeDtypeStruct((2, n), jnp.int32),
        ],
    )(hidden_states, w_router)

    srctok = _sc_dispatch(num_experts, capacity, n, po)
    srctok = srctok.reshape(num_experts, capacity, 1)

    x_bf = hidden_states.astype(jnp.bfloat16)

    tile = 512
    nt = ffn // tile
    d_all = pl.pallas_call(
        functools.partial(_ffn_body, capacity, nt),
        grid=(num_experts, nt),
        in_specs=[
            pl.BlockSpec((n, hidden), lambda e, f: (0, 0)),
            pl.BlockSpec((1, hidden, tile), lambda e, f: (e, 0, f)),
            pl.BlockSpec((1, hidden, tile), lambda e, f: (e, 0, f)),
            pl.BlockSpec((1, tile, hidden), lambda e, f: (e, f, 0)),
            pl.BlockSpec((1, capacity, 1), lambda e, f: (e, 0, 0)),
        ],
        out_specs=pl.BlockSpec((1, capacity, hidden), lambda e, f: (e, 0, 0)),
        out_shape=jax.ShapeDtypeStruct((num_experts, capacity, hidden),
                                       jnp.bfloat16),
        scratch_shapes=[
            pltpu.VMEM((capacity, n), jnp.bfloat16),
            pltpu.VMEM((capacity, hidden), jnp.bfloat16),
        ],
    )(x_bf, w_gate, w_up, w_down, srctok)

    cch = 512
    out = pl.pallas_call(
        functools.partial(_combine_body, capacity, num_experts),
        grid=(n // cch,),
        in_specs=[
            pl.BlockSpec((cch, 8), lambda c: (c, 0)),
            pl.BlockSpec((cch, 8), lambda c: (c, 0)),
            pl.BlockSpec((num_experts, capacity, hidden), lambda c: (0, 0, 0)),
        ],
        out_specs=pl.BlockSpec((cch, hidden), lambda c: (c, 0)),
        out_shape=jax.ShapeDtypeStruct((n, hidden), jnp.float32),
        scratch_shapes=[
            pltpu.VMEM((cch, capacity), jnp.bfloat16),
        ],
    )(mc, pc, d_all)

    return out, aux[0, 0]


# speedup vs baseline: 1.2470x; 1.0339x over previous
import dataclasses
import functools
import math

import jax
import jax.numpy as jnp
from jax.experimental import pallas as pl
from jax.experimental.pallas import tpu as pltpu
from jax.experimental.pallas import tpu_sc as plsc


def _sc_dispatch(num_experts, capacity, n, po):
    nslots = num_experts * capacity
    nflat = 2 * n

    cp = pltpu.CompilerParams()
    if "needs_layout_passes" in pltpu.CompilerParams.__dataclass_fields__:
        cp = dataclasses.replace(cp, needs_layout_passes=False)

    @pl.kernel(
        out_type=jax.ShapeDtypeStruct((nslots,), jnp.int32),
        compiler_params=cp,
        mesh=plsc.VectorSubcoreMesh(core_axis_name="core",
                                    subcore_axis_name="subcore"),
        scratch_types=[
            pltpu.VMEM((nflat,), jnp.int32),
            pltpu.VMEM((nslots + 16,), jnp.int32),
            pltpu.SemaphoreType.DMA,
        ],
    )
    def dispatch(po_hbm, out_hbm, pos_vmem, buf_vmem, sem):
        cid = jax.lax.axis_index("core")
        sid = jax.lax.axis_index("subcore")

        @pl.when((cid == 0) & (sid == 0))
        def _():
            pltpu.async_copy(po_hbm, pos_vmem, sem).wait()
            lane = jax.lax.iota(jnp.int32, 16)
            for i in range(nflat // 16):
                idx = pos_vmem[pl.ds(i * 16, 16)]
                tok = (lane + (i * 16)) & (n - 1)
                plsc.store_scatter(buf_vmem, [idx], tok)
            pltpu.async_copy(buf_vmem.at[pl.ds(0, nslots)], out_hbm,
                             sem).wait()

    return dispatch(po.reshape(-1))


def _router_body(num_experts, capacity, x_ref, wr_ref, mc_ref, mr_ref, pc_ref,
                 aux_ref, po_ref, xbf_ref):
    n = x_ref.shape[0]
    e_dim = num_experts
    x = x_ref[...]
    wr = wr_ref[...]
    xbf_ref[...] = x.astype(jnp.bfloat16)
    logits = jnp.dot(x, wr, preferred_element_type=jnp.float32)
    mx = jnp.max(logits, axis=-1, keepdims=True)
    ex = jnp.exp(logits - mx)
    probs = ex / jnp.sum(ex, axis=-1, keepdims=True)

    iota_e = jax.lax.broadcasted_iota(jnp.int32, (n, e_dim), 1)
    m1 = jnp.max(probs, axis=-1, keepdims=True)
    e1 = jnp.min(jnp.where(probs == m1, iota_e, e_dim), axis=-1, keepdims=True)
    probs2 = jnp.where(iota_e == e1, -1.0, probs)
    m2 = jnp.max(probs2, axis=-1, keepdims=True)
    e2 = jnp.min(jnp.where(probs2 == m2, iota_e, e_dim), axis=-1, keepdims=True)
    s = m1 + m2
    p1 = m1 / s
    p2 = m2 / s

    imp = jnp.sum(probs, axis=0, keepdims=True)
    load = jnp.sum((iota_e == e1).astype(jnp.float32)
                   + (iota_e == e2).astype(jnp.float32), axis=0, keepdims=True)
    aux_ref[...] = (e_dim * jnp.sum(imp * load, keepdims=True)) / (n * 2)

    e1f = e1.astype(jnp.float32)
    e2f = e2.astype(jnp.float32)
    zeros4 = jnp.zeros((n, 4), jnp.float32)
    packed = jnp.concatenate([e1f, e2f, p1, p2, zeros4], axis=1)
    packed_t = packed.T
    e1r = packed_t[0:1, :].astype(jnp.int32)
    e2r = packed_t[1:2, :].astype(jnp.int32)
    p1r = packed_t[2:3, :]
    p2r = packed_t[3:4, :]

    ch = 512
    tok_row = jax.lax.broadcasted_iota(jnp.int32, (1, n), 1)
    r1_parts = []
    r2_parts = []
    for c in range(n // ch):
        sl = slice(c * ch, (c + 1) * ch)
        tcol = jax.lax.broadcasted_iota(jnp.int32, (ch, 1), 0) + c * ch
        for k, (ei, pi) in enumerate(((e1[sl], p1[sl]), (e2[sl], p2[sl]))):
            cnt = jnp.zeros((ch, 1), jnp.int32)
            for kj, (ejr, pjr) in enumerate(((e1r, p1r), (e2r, p2r))):
                same = ejr == ei
                if kj < k:
                    earlier = tok_row <= tcol
                else:
                    earlier = tok_row < tcol
                beat = (pjr > pi) | ((pjr == pi) & earlier)
                cnt = cnt + jnp.sum((same & beat).astype(jnp.int32), axis=1,
                                    keepdims=True)
            if k == 0:
                r1_parts.append(cnt)
            else:
                r2_parts.append(cnt)
    r1 = jnp.concatenate(r1_parts, axis=0)
    r2 = jnp.concatenate(r2_parts, axis=0)

    zeros4i = jnp.zeros((n, 4), jnp.int32)
    mc_ref[...] = jnp.concatenate([e1, e2, r1, r2, zeros4i], axis=1)
    packed2 = jnp.concatenate([e1f, e2f, r1.astype(jnp.float32),
                               r2.astype(jnp.float32), zeros4], axis=1)
    mrt = packed2.T.astype(jnp.int32)
    mr_ref[...] = mrt
    nslots = num_experts * capacity
    pos1 = jnp.where(mrt[2:3, :] < capacity,
                     mrt[0:1, :] * capacity + mrt[2:3, :], nslots)
    pos2 = jnp.where(mrt[3:4, :] < capacity,
                     mrt[1:2, :] * capacity + mrt[3:4, :], nslots)
    po_ref[...] = jnp.concatenate([pos1, pos2], axis=0)
    pc_ref[...] = jnp.concatenate([p1, p2, zeros4, jnp.zeros((n, 2),
                                                             jnp.float32)],
                                  axis=1)


def _ffn_body(capacity, nt, x_ref, wg_ref, wu_ref, wd_ref, st_ref, d_ref,
              g_ref, seg_ref):
    e = pl.program_id(0)
    f = pl.program_id(1)
    n = x_ref.shape[0]
    ch = 512

    @pl.when(f == 0)
    def _build_seg():
        tok = st_ref[0]
        for c in range(n // ch):
            sl = slice(c * ch, (c + 1) * ch)
            iota_t = (jax.lax.broadcasted_iota(jnp.int32, (capacity, ch), 1)
                      + c * ch)
            g_ref[:, sl] = (tok == iota_t).astype(jnp.bfloat16)
        seg32 = jnp.dot(g_ref[...], x_ref[...],
                        preferred_element_type=jnp.float32)
        seg_ref[...] = seg32.astype(jnp.bfloat16)

    gate = jnp.dot(seg_ref[...], wg_ref[0].astype(jnp.bfloat16),
                   preferred_element_type=jnp.float32)
    up = jnp.dot(seg_ref[...], wu_ref[0].astype(jnp.bfloat16),
                 preferred_element_type=jnp.float32)
    h = (gate * jax.nn.sigmoid(gate)) * up
    dpart = jnp.dot(h.astype(jnp.bfloat16), wd_ref[0].astype(jnp.bfloat16),
                    preferred_element_type=jnp.float32)

    @pl.when(f == 0)
    def _init_d():
        d_ref[0] = dpart.astype(jnp.bfloat16)

    @pl.when(f > 0)
    def _acc_d():
        d_ref[0] = (d_ref[0].astype(jnp.float32) + dpart).astype(jnp.bfloat16)


def _combine_body(capacity, num_experts, mc_ref, pc_ref, d_ref, out_ref,
                  w_ref):
    ch = out_ref.shape[0]
    hidden = out_ref.shape[1]
    mc = mc_ref[...]
    pc = pc_ref[...]
    e1c = mc[:, 0:1]
    e2c = mc[:, 1:2]
    r1c = mc[:, 2:3]
    r2c = mc[:, 3:4]
    p1c = pc[:, 0:1]
    p2c = pc[:, 1:2]
    iota_rw = jax.lax.broadcasted_iota(jnp.int32, (ch, capacity), 1)
    acc = jnp.zeros((ch, hidden), jnp.float32)
    for e in range(num_experts):
        w = (jnp.where((e1c == e) & (iota_rw == r1c), p1c, 0.0)
             + jnp.where((e2c == e) & (iota_rw == r2c), p2c, 0.0))
        w_ref[...] = w.astype(jnp.bfloat16)
        acc = acc + jnp.dot(w_ref[...], d_ref[e],
                            preferred_element_type=jnp.float32)
    out_ref[...] = acc


def kernel(hidden_states, w_router, w_gate, w_up, w_down):
    n, hidden = hidden_states.shape
    num_experts = w_router.shape[1]
    ffn = w_gate.shape[2]
    capacity = max(4, math.ceil(1.25 * n * 2 / num_experts))
    capacity = min(capacity, n * 2)

    mc, mr, pc, aux, po, x_bf = pl.pallas_call(
        functools.partial(_router_body, num_experts, capacity),
        in_specs=[
            pl.BlockSpec((n, hidden), lambda: (0, 0)),
            pl.BlockSpec((hidden, num_experts), lambda: (0, 0)),
        ],
        out_specs=[
            pl.BlockSpec((n, 8), lambda: (0, 0)),
            pl.BlockSpec((8, n), lambda: (0, 0)),
            pl.BlockSpec((n, 8), lambda: (0, 0)),
            pl.BlockSpec((1, 1), lambda: (0, 0)),
            pl.BlockSpec((2, n), lambda: (0, 0)),
            pl.BlockSpec((n, hidden), lambda: (0, 0)),
        ],
        out_shape=[
            jax.ShapeDtypeStruct((n, 8), jnp.int32),
            jax.ShapeDtypeStruct((8, n), jnp.int32),
            jax.ShapeDtypeStruct((n, 8), jnp.float32),
            jax.ShapeDtypeStruct((1, 1), jnp.float32),
            jax.ShapeDtypeStruct((2, n), jnp.int32),
            jax.ShapeDtypeStruct((n, hidden), jnp.bfloat16),
        ],
    )(hidden_states, w_router)

    srctok = _sc_dispatch(num_experts, capacity, n, po)
    srctok = srctok.reshape(num_experts, capacity, 1)

    tile = 512
    nt = ffn // tile
    d_all = pl.pallas_call(
        functools.partial(_ffn_body, capacity, nt),
        grid=(num_experts, nt),
        in_specs=[
            pl.BlockSpec((n, hidden), lambda e, f: (0, 0)),
            pl.BlockSpec((1, hidden, tile), lambda e, f: (e, 0, f)),
            pl.BlockSpec((1, hidden, tile), lambda e, f: (e, 0, f)),
            pl.BlockSpec((1, tile, hidden), lambda e, f: (e, f, 0)),
            pl.BlockSpec((1, capacity, 1), lambda e, f: (e, 0, 0)),
        ],
        out_specs=pl.BlockSpec((1, capacity, hidden), lambda e, f: (e, 0, 0)),
        out_shape=jax.ShapeDtypeStruct((num_experts, capacity, hidden),
                                       jnp.bfloat16),
        scratch_shapes=[
            pltpu.VMEM((capacity, n), jnp.bfloat16),
            pltpu.VMEM((capacity, hidden), jnp.bfloat16),
        ],
    )(x_bf, w_gate, w_up, w_down, srctok)

    cch = 512
    out = pl.pallas_call(
        functools.partial(_combine_body, capacity, num_experts),
        grid=(n // cch,),
        in_specs=[
            pl.BlockSpec((cch, 8), lambda c: (c, 0)),
            pl.BlockSpec((cch, 8), lambda c: (c, 0)),
            pl.BlockSpec((num_experts, capacity, hidden), lambda c: (0, 0, 0)),
        ],
        out_specs=pl.BlockSpec((cch, hidden), lambda c: (c, 0)),
        out_shape=jax.ShapeDtypeStruct((n, hidden), jnp.float32),
        scratch_shapes=[
            pltpu.VMEM((cch, capacity), jnp.bfloat16),
        ],
    )(mc, pc, d_all)

    return out, aux[0, 0]
